# baseline (speedup 1.0000x reference)
import jax
import jax.numpy as jnp
from jax.experimental import pallas as pl


def _scale_body(logits_ref, temp_ref, out_ref):
    inv = 1.0 + jnp.exp(-temp_ref[...])
    out_ref[...] = logits_ref[...] * inv


def kernel(logits, temp):
    B, V = logits.shape
    BV = 16384
    grid = (pl.cdiv(V, BV),)
    return pl.pallas_call(
        _scale_body,
        grid=grid,
        in_specs=[
            pl.BlockSpec((B, BV), lambda i: (0, i)),
            pl.BlockSpec((B, 1), lambda i: (0, 0)),
        ],
        out_specs=pl.BlockSpec((B, BV), lambda i: (0, i)),
        out_shape=jax.ShapeDtypeStruct((B, V), logits.dtype),
    )(logits, temp)


# speedup vs baseline: 1.0271x; 1.0271x over previous
import jax
import jax.numpy as jnp
from jax.experimental import pallas as pl


def _scale_body(logits_ref, temp_ref, out_ref):
    inv = 1.0 + jnp.exp(-temp_ref[...])
    out_ref[...] = logits_ref[...] * inv


def kernel(logits, temp):
    B, V = logits.shape
    BV = 57344
    grid = (pl.cdiv(V, BV),)
    return pl.pallas_call(
        _scale_body,
        grid=grid,
        in_specs=[
            pl.BlockSpec((B, BV), lambda i: (0, i)),
            pl.BlockSpec((B, 1), lambda i: (0, 0)),
        ],
        out_specs=pl.BlockSpec((B, BV), lambda i: (0, i)),
        out_shape=jax.ShapeDtypeStruct((B, V), logits.dtype),
    )(logits, temp)
